# baseline (speedup 1.0000x reference)
import functools

import jax
import jax.numpy as jnp
from jax import lax
from jax.experimental import pallas as pl
from jax.experimental.pallas import tpu as pltpu
from jax.experimental.pallas import tpu_sc as plsc

B = 64
S = 1024
D = 768
N = B * S
L = 16
KD = D // L

NUM_WORKERS = 32
PPW = S // NUM_WORKERS
ROWJ = 8


def _add_pos(rows_v, pos_v, posrow_v, buf, t):
    rows2d = rows_v.at[buf]
    for k in range(KD):
        sl = pl.ds(k * L, L)
        posrow_v[sl] = pos_v[t, sl]

    def body(j, carry):
        for k in range(KD):
            sl = pl.ds(k * L, L)
            plsc.addupdate(rows2d.at[j, sl], posrow_v[sl])
        return carry

    lax.fori_loop(0, B, body, 0)


def _store_out_idx(out_idx_v, buf, wbase, t):
    for m in range(B // L):
        vec = (lax.iota(jnp.int32, L) + (m * L)) * S + (wbase + t)
        out_idx_v[buf, pl.ds(m * L, L)] = vec


def _emb_body(xt_hbm, tok_hbm, pos_hbm, out_hbm,
              idx_v, rows_v, pos_v, posrow_v, out_idx_v, g0, g1, o0, o1):
    wid = lax.axis_index("s") * 2 + lax.axis_index("c")
    wbase = wid * PPW
    gsem = (g0, g1)
    osem = (o0, o1)

    pltpu.sync_copy(xt_hbm.at[pl.ds(wbase * B, PPW * B)], idx_v)
    pltpu.sync_copy(pos_hbm.at[pl.ds(wbase, PPW)], pos_v)

    def issue_gather(t, buf):
        pltpu.async_copy(tok_hbm.at[idx_v.at[pl.ds(t * B, B)]],
                         rows_v.at[buf], gsem[buf])

    def wait_gather(buf):
        pltpu.make_async_copy(tok_hbm.at[idx_v.at[pl.ds(0, B)]],
                              rows_v.at[buf], gsem[buf]).wait()

    def issue_store(buf):
        pltpu.async_copy(rows_v.at[buf], out_hbm.at[out_idx_v.at[buf]],
                         osem[buf])

    def wait_store(buf):
        pltpu.make_async_copy(rows_v.at[buf], out_hbm.at[out_idx_v.at[buf]],
                              osem[buf]).wait()

    issue_gather(0, 0)
    issue_gather(1, 1)
    wait_gather(0)
    _add_pos(rows_v, pos_v, posrow_v, 0, 0)
    _store_out_idx(out_idx_v, 0, wbase, 0)
    issue_store(0)

    def pair(tt, carry):
        t = 2 * tt + 1
        wait_gather(1)
        _add_pos(rows_v, pos_v, posrow_v, 1, t)
        _store_out_idx(out_idx_v, 1, wbase, t)
        issue_store(1)
        wait_store(0)
        issue_gather(t + 1, 0)

        wait_gather(0)
        _add_pos(rows_v, pos_v, posrow_v, 0, t + 1)
        _store_out_idx(out_idx_v, 0, wbase, t + 1)
        issue_store(0)
        wait_store(1)
        issue_gather(t + 2, 1)
        return carry

    lax.fori_loop(0, (PPW - 2) // 2, pair, 0)

    wait_gather(1)
    _add_pos(rows_v, pos_v, posrow_v, 1, PPW - 1)
    wait_store(0)
    _store_out_idx(out_idx_v, 1, wbase, PPW - 1)
    issue_store(1)
    wait_store(1)


@jax.jit
def _emb(x_t, token_emb, pos2d):
    mesh = plsc.VectorSubcoreMesh(core_axis_name="c", subcore_axis_name="s")
    f = functools.partial(
        pl.kernel,
        out_type=jax.ShapeDtypeStruct((N, D), jnp.float32),
        mesh=mesh,
        scratch_types=[
            pltpu.VMEM((PPW * B,), jnp.int32),
            pltpu.VMEM((2, B, D), jnp.float32),
            pltpu.VMEM((PPW, D), jnp.float32),
            pltpu.VMEM((D,), jnp.float32),
            pltpu.VMEM((2, B), jnp.int32),
            pltpu.SemaphoreType.DMA,
            pltpu.SemaphoreType.DMA,
            pltpu.SemaphoreType.DMA,
            pltpu.SemaphoreType.DMA,
        ],
    )(_emb_body)
    return f(x_t, token_emb, pos2d)


def kernel(x, token_emb, pos_emb):
    x_t = x.T.reshape(N)
    pos2d = pos_emb.reshape(S, D)
    out = _emb(x_t, token_emb, pos2d)
    return out.reshape(B, S, D)


# speedup vs baseline: 1.1449x; 1.1449x over previous
import functools

import jax
import jax.numpy as jnp
from jax import lax
from jax.experimental import pallas as pl
from jax.experimental.pallas import tpu as pltpu
from jax.experimental.pallas import tpu_sc as plsc

B = 64
S = 1024
D = 768
N = B * S
L = 16
KD = D // L

NUM_WORKERS = 32
PPW = S // NUM_WORKERS
ROWJ = 8


def _add_pos(rows2d, pos_v, posrow_v, t):
    for k in range(KD):
        sl = pl.ds(k * L, L)
        posrow_v[sl] = pos_v[t, sl]

    def body(j, carry):
        for k in range(KD):
            sl = pl.ds(k * L, L)
            plsc.addupdate(rows2d.at[j, sl], posrow_v[sl])
        return carry

    lax.fori_loop(0, B, body, 0)


def _store_out_idx(oidx, wbase, t):
    for m in range(B // L):
        vec = (lax.iota(jnp.int32, L) + (m * L)) * S + (wbase + t)
        oidx[pl.ds(m * L, L)] = vec


def _emb_body(xt_hbm, tok_hbm, pos_hbm, out_hbm,
              idx_v, rows0_v, rows1_v, pos_v, posrow_v, oidx0_v, oidx1_v,
              g0, g1, o0, o1):
    wid = lax.axis_index("s") * 2 + lax.axis_index("c")
    wbase = wid * PPW
    rows = (rows0_v, rows1_v)
    oidx = (oidx0_v, oidx1_v)
    gsem = (g0, g1)
    osem = (o0, o1)

    pltpu.sync_copy(xt_hbm.at[pl.ds(wbase * B, PPW * B)], idx_v)
    pltpu.sync_copy(pos_hbm.at[pl.ds(wbase, PPW)], pos_v)

    def issue_gather(t, buf):
        pltpu.async_copy(tok_hbm.at[idx_v.at[pl.ds(t * B, B)]],
                         rows[buf], gsem[buf])

    def wait_gather(buf):
        pltpu.make_async_copy(tok_hbm.at[idx_v.at[pl.ds(0, B)]],
                              rows[buf], gsem[buf]).wait()

    def issue_store(buf):
        pltpu.async_copy(rows[buf], out_hbm.at[oidx[buf]], osem[buf])

    def wait_store(buf):
        pltpu.make_async_copy(rows[buf], out_hbm.at[oidx[buf]],
                              osem[buf]).wait()

    issue_gather(0, 0)
    issue_gather(1, 1)
    wait_gather(0)
    _add_pos(rows0_v, pos_v, posrow_v, 0)
    _store_out_idx(oidx0_v, wbase, 0)
    issue_store(0)

    def pair(tt, carry):
        t = 2 * tt + 1
        wait_gather(1)
        wait_store(0)
        issue_gather(t + 1, 0)
        _add_pos(rows1_v, pos_v, posrow_v, t)
        _store_out_idx(oidx1_v, wbase, t)
        issue_store(1)

        wait_gather(0)
        wait_store(1)
        issue_gather(t + 2, 1)
        _add_pos(rows0_v, pos_v, posrow_v, t + 1)
        _store_out_idx(oidx0_v, wbase, t + 1)
        issue_store(0)
        return carry

    lax.fori_loop(0, (PPW - 2) // 2, pair, 0)

    wait_gather(1)
    _add_pos(rows1_v, pos_v, posrow_v, PPW - 1)
    wait_store(0)
    _store_out_idx(oidx1_v, wbase, PPW - 1)
    issue_store(1)
    wait_store(1)


@jax.jit
def _emb(x_t, token_emb, pos2d):
    mesh = plsc.VectorSubcoreMesh(core_axis_name="c", subcore_axis_name="s")
    f = functools.partial(
        pl.kernel,
        out_type=jax.ShapeDtypeStruct((N, D), jnp.float32),
        mesh=mesh,
        scratch_types=[
            pltpu.VMEM((PPW * B,), jnp.int32),
            pltpu.VMEM((B, D), jnp.float32),
            pltpu.VMEM((B, D), jnp.float32),
            pltpu.VMEM((PPW, D), jnp.float32),
            pltpu.VMEM((D,), jnp.float32),
            pltpu.VMEM((B,), jnp.int32),
            pltpu.VMEM((B,), jnp.int32),
            pltpu.SemaphoreType.DMA,
            pltpu.SemaphoreType.DMA,
            pltpu.SemaphoreType.DMA,
            pltpu.SemaphoreType.DMA,
        ],
    )(_emb_body)
    return f(x_t, token_emb, pos2d)


def kernel(x, token_emb, pos_emb):
    x_t = x.T.reshape(N)
    pos2d = pos_emb.reshape(S, D)
    out = _emb(x_t, token_emb, pos2d)
    return out.reshape(B, S, D)


# speedup vs baseline: 1.1591x; 1.0123x over previous
import functools

import jax
import jax.numpy as jnp
from jax import lax
from jax.experimental import pallas as pl
from jax.experimental.pallas import tpu as pltpu
from jax.experimental.pallas import tpu_sc as plsc

B = 64
S = 1024
D = 768
N = B * S
L = 16

NUM_WORKERS = 32
PER_W = N // NUM_WORKERS
REPS = PER_W // S
C = 32
NPCHUNKS = S // C
NSTEPS = NPCHUNKS * REPS


def _emb_body(x_hbm, tok_hbm, pos_hbm, out_hbm,
              idx_v, rows_v, pos_v, gsem, osem, psem):
    wid = lax.axis_index("s") * 2 + lax.axis_index("c")
    base = wid * PER_W

    pltpu.sync_copy(x_hbm.at[pl.ds(base, PER_W)], idx_v)
    pltpu.sync_copy(pos_hbm.at[pl.ds(0, C)], pos_v.at[0])
    pltpu.async_copy(tok_hbm.at[idx_v.at[pl.ds(0, C)]], rows_v.at[0], gsem)

    def step(s, carry):
        b = s & 1
        nb = 1 - b
        c = s >> 1
        pb = c & 1
        p0 = c * C
        start = base + b * S + p0

        @pl.when(s + 1 < NSTEPS)
        def _():
            @pl.when(s >= 1)
            def _():
                pltpu.make_async_copy(
                    rows_v.at[nb], out_hbm.at[pl.ds(0, C)], osem).wait()
            c1 = (s + 1) >> 1
            off1 = nb * S + c1 * C
            pltpu.async_copy(
                tok_hbm.at[idx_v.at[pl.ds(off1, C)]], rows_v.at[nb], gsem)

        @pl.when((b == 0) & (c + 1 < NPCHUNKS))
        def _():
            pltpu.async_copy(
                pos_hbm.at[pl.ds(p0 + C, C)], pos_v.at[1 - pb], psem)

        pltpu.make_async_copy(
            tok_hbm.at[idx_v.at[pl.ds(0, C)]], rows_v.at[b], gsem).wait()

        @pl.when((b == 0) & (s > 0))
        def _():
            pltpu.make_async_copy(
                pos_hbm.at[pl.ds(0, C)], pos_v.at[pb], psem).wait()

        def addrow(j, carry2):
            for k in range(D // L):
                sl = pl.ds(k * L, L)
                plsc.addupdate(rows_v.at[b, j, sl], pos_v[pb, j, sl])
            return carry2

        lax.fori_loop(0, C, addrow, 0)

        pltpu.async_copy(rows_v.at[b], out_hbm.at[pl.ds(start, C)], osem)
        return carry

    lax.fori_loop(0, NSTEPS, step, 0)

    pltpu.make_async_copy(rows_v.at[0], out_hbm.at[pl.ds(0, C)], osem).wait()
    pltpu.make_async_copy(rows_v.at[1], out_hbm.at[pl.ds(0, C)], osem).wait()


@jax.jit
def _emb(x_flat, token_emb, pos2d):
    mesh = plsc.VectorSubcoreMesh(core_axis_name="c", subcore_axis_name="s")
    f = functools.partial(
        pl.kernel,
        out_type=jax.ShapeDtypeStruct((N, D), jnp.float32),
        mesh=mesh,
        scratch_types=[
            pltpu.VMEM((PER_W,), jnp.int32),
            pltpu.VMEM((2, C, D), jnp.float32),
            pltpu.VMEM((2, C, D), jnp.float32),
            pltpu.SemaphoreType.DMA,
            pltpu.SemaphoreType.DMA,
            pltpu.SemaphoreType.DMA,
        ],
    )(_emb_body)
    return f(x_flat, token_emb, pos2d)


def kernel(x, token_emb, pos_emb):
    x_flat = x.reshape(N)
    pos2d = pos_emb.reshape(S, D)
    out = _emb(x_flat, token_emb, pos2d)
    return out.reshape(B, S, D)


# speedup vs baseline: 1.2837x; 1.1075x over previous
import functools

import jax
import jax.numpy as jnp
from jax import lax
from jax.experimental import pallas as pl
from jax.experimental.pallas import tpu as pltpu
from jax.experimental.pallas import tpu_sc as plsc

B = 64
S = 1024
D = 768
N = B * S
L = 16

NUM_WORKERS = 32
PER_W = N // NUM_WORKERS
REPS = PER_W // S
C = 64
NPCHUNKS = S // C


def _emb_body(x_hbm, tok_hbm, pos_hbm, out_hbm, idx_v, rows_v, pos_v, sem):
    wid = lax.axis_index("s") * 2 + lax.axis_index("c")
    base = wid * PER_W

    def pchunk(c, carry):
        p0 = c * C
        pltpu.sync_copy(pos_hbm.at[pl.ds(p0, C)], pos_v)
        for r in range(REPS):
            start = base + r * S + p0
            pltpu.sync_copy(x_hbm.at[pl.ds(start, C)], idx_v)
            pltpu.async_copy(tok_hbm.at[idx_v], rows_v, sem).wait()

            def addrow(j, carry2):
                for k in range(D // L):
                    sl = pl.ds(k * L, L)
                    plsc.addupdate(rows_v.at[j, sl], pos_v[j, sl])
                return carry2

            lax.fori_loop(0, C, addrow, 0)
            pltpu.sync_copy(rows_v, out_hbm.at[pl.ds(start, C)])
        return carry

    lax.fori_loop(0, NPCHUNKS, pchunk, 0)


@jax.jit
def _emb(x_flat, token_emb, pos2d):
    mesh = plsc.VectorSubcoreMesh(core_axis_name="c", subcore_axis_name="s")
    f = functools.partial(
        pl.kernel,
        out_type=jax.ShapeDtypeStruct((N, D), jnp.float32),
        mesh=mesh,
        scratch_types=[
            pltpu.VMEM((C,), jnp.int32),
            pltpu.VMEM((C, D), jnp.float32),
            pltpu.VMEM((C, D), jnp.float32),
            pltpu.SemaphoreType.DMA,
        ],
    )(_emb_body)
    return f(x_flat, token_emb, pos2d)


def kernel(x, token_emb, pos_emb):
    x_flat = x.reshape(N)
    pos2d = pos_emb.reshape(S, D)
    out = _emb(x_flat, token_emb, pos2d)
    return out.reshape(B, S, D)
